# baseline (speedup 1.0000x reference)
import functools

import jax
import jax.numpy as jnp
from jax import lax
from jax.experimental import pallas as pl
from jax.experimental.pallas import tpu as pltpu
from jax.experimental.pallas import tpu_sc as plsc

NC = 2
NS = 16
NW = NC * NS
CHUNK = 80
NBUF = 2
ISN = 6
DDEPTH = 3
BLK = 2000

_mesh = functools.partial(
    plsc.VectorSubcoreMesh, core_axis_name="c", subcore_axis_name="s",
    num_cores=NC, num_subcores=NS)

_Z16 = functools.partial(jnp.zeros, (16,), jnp.float32)


def _sc_degrees(ei, n_acc, nchunk):
  slab_n = n_acc // NS

  @functools.partial(
      pl.kernel,
      out_type=jax.ShapeDtypeStruct((NC, 2, n_acc), jnp.float32),
      mesh=_mesh(),
      scratch_types=[
          pltpu.VMEM((2, nchunk, CHUNK), jnp.int32),
          pltpu.VMEM((CHUNK,), jnp.float32),
          pltpu.VMEM((slab_n,), jnp.float32),
          pltpu.VMEM_SHARED((n_acc,), jnp.float32),
          pltpu.VMEM_SHARED((n_acc,), jnp.float32),
          pltpu.SemaphoreType.DMA((DDEPTH + 1,)),
          pltpu.SemaphoreType.DMA((DDEPTH + 1,)),
      ],
  )
  def k(ei_hbm, deg_hbm, idx_v, ones_v, zb_v, dego_sh, degi_sh, osem, isem):
    cid = lax.axis_index("c")
    sid = lax.axis_index("s")
    w = cid * NS + sid
    pltpu.sync_copy(ei_hbm.at[0, w], idx_v.at[0])
    pltpu.sync_copy(ei_hbm.at[1, w], idx_v.at[1])
    for t in range(CHUNK // 16):
      ones_v[pl.ds(t * 16, 16)] = jnp.ones((16,), jnp.float32)

    def zfill(t, carry):
      zb_v[pl.ds(t * 16, 16)] = _Z16()
      return carry

    lax.fori_loop(0, slab_n // 16, zfill, 0)
    slab = pl.ds(sid * slab_n, slab_n)
    pltpu.sync_copy(zb_v, dego_sh.at[slab])
    pltpu.sync_copy(zb_v, degi_sh.at[slab])
    plsc.subcore_barrier()

    def fire(c):
      cb = lax.rem(c, DDEPTH + 1)
      pltpu.async_copy(ones_v, dego_sh.at[idx_v.at[0, c]], osem.at[cb],
                       add=True)
      pltpu.async_copy(ones_v, degi_sh.at[idx_v.at[1, c]], isem.at[cb],
                       add=True)

    def drain(c):
      cb = lax.rem(c, DDEPTH + 1)
      pltpu.make_async_copy(
          ones_v, dego_sh.at[idx_v.at[0, c]], osem.at[cb]).wait()
      pltpu.make_async_copy(
          ones_v, degi_sh.at[idx_v.at[1, c]], isem.at[cb]).wait()

    for c in range(DDEPTH):
      fire(c)

    def body(j, carry):
      @pl.when(j + DDEPTH < nchunk)
      def _():
        fire(j + DDEPTH)

      drain(j)
      return carry

    lax.fori_loop(0, nchunk, body, 0)
    plsc.subcore_barrier()
    pltpu.sync_copy(dego_sh.at[slab], deg_hbm.at[cid, 0, slab])
    pltpu.sync_copy(degi_sh.at[slab], deg_hbm.at[cid, 1, slab])

  return k(ei)


def _sc_aggregate(ei, h_scaled, n_acc, nchunk, d):
  slab_n = n_acc // NS

  @functools.partial(
      pl.kernel,
      out_type=jax.ShapeDtypeStruct((NC, n_acc, d), jnp.float32),
      mesh=_mesh(),
      scratch_types=[
          pltpu.VMEM((nchunk, CHUNK), jnp.int32),
          pltpu.VMEM((ISN, CHUNK), jnp.int32),
          pltpu.VMEM((NBUF, CHUNK, d), jnp.float32),
          pltpu.VMEM((CHUNK, d), jnp.float32),
          pltpu.VMEM_SHARED((n_acc, d), jnp.float32),
          pltpu.SemaphoreType.DMA((NBUF,)),
          pltpu.SemaphoreType.DMA((NBUF,)),
          pltpu.SemaphoreType.DMA((ISN,)),
      ],
  )
  def k(ei_hbm, h_hbm, agg_hbm,
        dst_v, src_v, rows_v, zb_v, agg_sh, gsem, ssem, xsem):
    cid = lax.axis_index("c")
    sid = lax.axis_index("s")
    w = cid * NS + sid
    pltpu.sync_copy(ei_hbm.at[1, w], dst_v)

    def zfill(t, carry):
      for u in range(d // 128):
        zb_v[t, pl.ds(u * 128, 128)] = jnp.zeros((128,), jnp.float32)
      return carry

    lax.fori_loop(0, CHUNK, zfill, 0)
    for z in range(slab_n // CHUNK):
      pltpu.sync_copy(
          zb_v, agg_sh.at[pl.ds(sid * slab_n + z * CHUNK, CHUNK)])
    plsc.subcore_barrier()

    def fire_idx(r, rb):
      pltpu.async_copy(ei_hbm.at[0, w, r], src_v.at[rb], xsem.at[rb])

    def fire_gather(r, rb, bb):
      pltpu.make_async_copy(
          ei_hbm.at[0, w, r], src_v.at[rb], xsem.at[rb]).wait()
      pltpu.async_copy(h_hbm.at[src_v.at[rb]], rows_v.at[bb], gsem.at[bb])

    for r in range(min(DDEPTH, nchunk)):
      fire_idx(r, r)
    for bi in range(min(NBUF, nchunk)):
      fire_gather(bi, bi, bi)

    def body(j, carry):
      jb = lax.rem(j, NBUF)

      @pl.when(j + DDEPTH < nchunk)
      def _():
        fire_idx(j + DDEPTH, lax.rem(j + DDEPTH, ISN))

      pltpu.make_async_copy(
          h_hbm.at[src_v.at[lax.rem(j, ISN)]], rows_v.at[jb],
          gsem.at[jb]).wait()
      pltpu.async_copy(
          rows_v.at[jb], agg_sh.at[dst_v.at[j]], ssem.at[jb], add=True)

      @pl.when(j >= 1)
      def _():
        jp = j - 1
        pb = lax.rem(jp, NBUF)
        pltpu.make_async_copy(
            rows_v.at[pb], agg_sh.at[dst_v.at[jp]], ssem.at[pb]).wait()

        @pl.when(jp + NBUF < nchunk)
        def _():
          jn = jp + NBUF
          fire_gather(jn, lax.rem(jn, ISN), pb)

      return carry

    lax.fori_loop(0, nchunk, body, 0)
    pltpu.make_async_copy(
        rows_v.at[(nchunk - 1) % NBUF], agg_sh.at[dst_v.at[nchunk - 1]],
        ssem.at[(nchunk - 1) % NBUF]).wait()
    plsc.subcore_barrier()
    slab = pl.ds(sid * slab_n, slab_n)
    pltpu.sync_copy(agg_sh.at[slab], agg_hbm.at[cid, slab])

  return k(ei, h_scaled)


def _tc_matmul_scale(x, w, degp, n, d):
  grid = n // BLK

  def body(x_ref, w_ref, deg_ref, o_ref):
    deg = deg_ref[0, 0] + deg_ref[1, 0]
    norm = lax.rsqrt(jnp.maximum(deg, 1.0))
    h = jnp.dot(x_ref[...], w_ref[...], preferred_element_type=jnp.float32)
    o_ref[...] = h * norm

  return pl.pallas_call(
      body,
      grid=(grid,),
      in_specs=[
          pl.BlockSpec((BLK, d), lambda i: (i, 0)),
          pl.BlockSpec((d, d), lambda i: (0, 0)),
          pl.BlockSpec((NC, 2, BLK, 1), lambda i: (0, 0, i, 0)),
      ],
      out_specs=pl.BlockSpec((BLK, d), lambda i: (i, 0)),
      out_shape=jax.ShapeDtypeStruct((n, d), jnp.float32),
  )(x, w, degp)


def _tc_finalize(aggp, degp, b2, n, d):
  grid = n // BLK

  def body(agg_ref, deg_ref, b_ref, act_ref, clone_ref):
    agg = agg_ref[0] + agg_ref[1]
    deg = deg_ref[0, 1] + deg_ref[1, 1]
    norm = lax.rsqrt(jnp.maximum(deg, 1.0))
    out = agg * norm + b_ref[...]
    act = jnp.maximum(out, 0.0)
    act_ref[...] = act
    clone_ref[...] = jnp.where(act >= 0.5, 1.0, 0.0).astype(jnp.float32)

  return pl.pallas_call(
      body,
      grid=(grid,),
      in_specs=[
          pl.BlockSpec((NC, BLK, d), lambda i: (0, i, 0)),
          pl.BlockSpec((NC, 2, BLK, 1), lambda i: (0, 0, i, 0)),
          pl.BlockSpec((1, d), lambda i: (0, 0)),
      ],
      out_specs=[
          pl.BlockSpec((BLK, d), lambda i: (i, 0)),
          pl.BlockSpec((BLK, d), lambda i: (i, 0)),
      ],
      out_shape=[
          jax.ShapeDtypeStruct((n, d), jnp.float32),
          jax.ShapeDtypeStruct((n, d), jnp.float32),
      ],
  )(aggp, degp, b2)


def kernel(in_feat, edge_index, W, b):
  n, d = in_feat.shape
  e = edge_index.shape[1]
  assert e % (NW * CHUNK) == 0 and n % BLK == 0
  nchunk = e // (NW * CHUNK)
  n_acc = NS * CHUNK * (-(-n // (NS * CHUNK)))

  ei = edge_index.astype(jnp.int32).reshape(2, NW, nchunk, CHUNK)

  degp = _sc_degrees(ei, n_acc, nchunk)
  degp4 = degp.reshape(NC, 2, n_acc, 1)
  h_scaled = _tc_matmul_scale(in_feat, W, degp4, n, d)
  aggp = _sc_aggregate(ei, h_scaled, n_acc, nchunk, d)
  h_act, h_clone = _tc_finalize(aggp, degp4, b.reshape(1, d), n, d)
  return (h_act, h_clone)


# speedup vs baseline: 1.0577x; 1.0577x over previous
import functools

import jax
import jax.numpy as jnp
from jax import lax
from jax.experimental import pallas as pl
from jax.experimental.pallas import tpu as pltpu
from jax.experimental.pallas import tpu_sc as plsc

NC = 2
NS = 16
NW = NC * NS
CHUNK = 128
NBUF = 2
ISN = 6
DDEPTH = 3
BLK = 2000

_mesh = functools.partial(
    plsc.VectorSubcoreMesh, core_axis_name="c", subcore_axis_name="s",
    num_cores=NC, num_subcores=NS)


def _chunk_range(w, ncht):
  c0 = (w * ncht) // NW
  c1 = ((w + 1) * ncht) // NW
  return c0, c1


def _sc_degrees(es, ed, zeros1, n_acc, ncht):
  slab_n = n_acc // NS

  @functools.partial(
      pl.kernel,
      out_type=jax.ShapeDtypeStruct((NC, 2, n_acc), jnp.float32),
      mesh=_mesh(),
      scratch_types=[
          pltpu.VMEM((ISN, CHUNK), jnp.int32),
          pltpu.VMEM((ISN, CHUNK), jnp.int32),
          pltpu.VMEM((CHUNK,), jnp.float32),
          pltpu.VMEM_SHARED((n_acc,), jnp.float32),
          pltpu.VMEM_SHARED((n_acc,), jnp.float32),
          pltpu.SemaphoreType.DMA((ISN,)),
          pltpu.SemaphoreType.DMA((ISN,)),
          pltpu.SemaphoreType.DMA((DDEPTH + 1,)),
          pltpu.SemaphoreType.DMA((DDEPTH + 1,)),
      ],
  )
  def k(es_hbm, ed_hbm, z_hbm, deg_hbm, sidx_v, didx_v, ones_v,
        dego_sh, degi_sh, xsem, ysem, osem, isem):
    cid = lax.axis_index("c")
    sid = lax.axis_index("s")
    w = cid * NS + sid
    c0, c1 = _chunk_range(w, ncht)
    nloc = c1 - c0
    for t in range(CHUNK // 16):
      ones_v[pl.ds(t * 16, 16)] = jnp.ones((16,), jnp.float32)
    slab = pl.ds(sid * slab_n, slab_n)
    pltpu.sync_copy(z_hbm.at[slab], dego_sh.at[slab])
    pltpu.sync_copy(z_hbm.at[slab], degi_sh.at[slab])
    plsc.subcore_barrier()

    def fire_idx(i, ib):
      off = pl.multiple_of((c0 + i) * CHUNK, CHUNK)
      pltpu.async_copy(es_hbm.at[pl.ds(off, CHUNK)], sidx_v.at[ib],
                       xsem.at[ib])
      pltpu.async_copy(ed_hbm.at[pl.ds(off, CHUNK)], didx_v.at[ib],
                       ysem.at[ib])

    def wait_idx(i, ib):
      off = pl.multiple_of((c0 + i) * CHUNK, CHUNK)
      pltpu.make_async_copy(
          es_hbm.at[pl.ds(off, CHUNK)], sidx_v.at[ib], xsem.at[ib]).wait()
      pltpu.make_async_copy(
          ed_hbm.at[pl.ds(off, CHUNK)], didx_v.at[ib], ysem.at[ib]).wait()

    def fire_sc(i):
      cb = lax.rem(i, DDEPTH + 1)
      ib = lax.rem(i, ISN)
      pltpu.async_copy(ones_v, dego_sh.at[sidx_v.at[ib]], osem.at[cb],
                       add=True)
      pltpu.async_copy(ones_v, degi_sh.at[didx_v.at[ib]], isem.at[cb],
                       add=True)

    def drain_sc(i):
      cb = lax.rem(i, DDEPTH + 1)
      ib = lax.rem(i, ISN)
      pltpu.make_async_copy(
          ones_v, dego_sh.at[sidx_v.at[ib]], osem.at[cb]).wait()
      pltpu.make_async_copy(
          ones_v, degi_sh.at[didx_v.at[ib]], isem.at[cb]).wait()

    for r in range(DDEPTH):
      fire_idx(r, r)

    def body(i, carry):
      @pl.when(i + DDEPTH < nloc)
      def _():
        fire_idx(i + DDEPTH, lax.rem(i + DDEPTH, ISN))

      wait_idx(i, lax.rem(i, ISN))
      fire_sc(i)

      @pl.when(i >= 2)
      def _():
        drain_sc(i - 2)

      return carry

    lax.fori_loop(0, nloc, body, 0)
    drain_sc(nloc - 2)
    drain_sc(nloc - 1)
    plsc.subcore_barrier()
    pltpu.sync_copy(dego_sh.at[slab], deg_hbm.at[cid, 0, slab])
    pltpu.sync_copy(degi_sh.at[slab], deg_hbm.at[cid, 1, slab])

  return k(es, ed, zeros1)


def _sc_aggregate(es, ed, h_scaled, zeros2, n_acc, ncht, d):
  slab_n = n_acc // NS

  @functools.partial(
      pl.kernel,
      out_type=jax.ShapeDtypeStruct((NC, n_acc, d), jnp.float32),
      mesh=_mesh(),
      scratch_types=[
          pltpu.VMEM((ISN, CHUNK), jnp.int32),
          pltpu.VMEM((ISN, CHUNK), jnp.int32),
          pltpu.VMEM((NBUF, CHUNK, d), jnp.float32),
          pltpu.VMEM_SHARED((n_acc, d), jnp.float32),
          pltpu.SemaphoreType.DMA((ISN,)),
          pltpu.SemaphoreType.DMA((ISN,)),
          pltpu.SemaphoreType.DMA((NBUF,)),
          pltpu.SemaphoreType.DMA((NBUF,)),
      ],
  )
  def k(es_hbm, ed_hbm, h_hbm, z_hbm, agg_hbm,
        sidx_v, didx_v, rows_v, agg_sh, xsem, ysem, gsem, ssem):
    cid = lax.axis_index("c")
    sid = lax.axis_index("s")
    w = cid * NS + sid
    c0, c1 = _chunk_range(w, ncht)
    nloc = c1 - c0
    slab = pl.ds(sid * slab_n, slab_n)
    pltpu.sync_copy(z_hbm.at[slab], agg_sh.at[slab])
    plsc.subcore_barrier()

    def fire_idx(i, ib):
      off = pl.multiple_of((c0 + i) * CHUNK, CHUNK)
      pltpu.async_copy(es_hbm.at[pl.ds(off, CHUNK)], sidx_v.at[ib],
                       xsem.at[ib])
      pltpu.async_copy(ed_hbm.at[pl.ds(off, CHUNK)], didx_v.at[ib],
                       ysem.at[ib])

    def wait_idx(i, ib):
      off = pl.multiple_of((c0 + i) * CHUNK, CHUNK)
      pltpu.make_async_copy(
          es_hbm.at[pl.ds(off, CHUNK)], sidx_v.at[ib], xsem.at[ib]).wait()
      pltpu.make_async_copy(
          ed_hbm.at[pl.ds(off, CHUNK)], didx_v.at[ib], ysem.at[ib]).wait()

    def fire_gather(i, bb):
      ib = lax.rem(i, ISN)
      wait_idx(i, ib)
      pltpu.async_copy(h_hbm.at[sidx_v.at[ib]], rows_v.at[bb], gsem.at[bb])

    for r in range(DDEPTH):
      fire_idx(r, r)
    for bi in range(NBUF):
      fire_gather(bi, bi)

    def body(i, carry):
      ib = lax.rem(i, ISN)
      jb = lax.rem(i, NBUF)

      @pl.when(i + DDEPTH < nloc)
      def _():
        fire_idx(i + DDEPTH, lax.rem(i + DDEPTH, ISN))

      pltpu.make_async_copy(
          h_hbm.at[sidx_v.at[ib]], rows_v.at[jb], gsem.at[jb]).wait()
      pltpu.async_copy(
          rows_v.at[jb], agg_sh.at[didx_v.at[ib]], ssem.at[jb], add=True)

      @pl.when(i >= 1)
      def _():
        ip = i - 1
        pb = lax.rem(ip, NBUF)
        pltpu.make_async_copy(
            rows_v.at[pb], agg_sh.at[didx_v.at[lax.rem(ip, ISN)]],
            ssem.at[pb]).wait()

        @pl.when(ip + NBUF < nloc)
        def _():
          fire_gather(ip + NBUF, pb)

      return carry

    lax.fori_loop(0, nloc, body, 0)
    pltpu.make_async_copy(
        rows_v.at[lax.rem(nloc - 1, NBUF)],
        agg_sh.at[didx_v.at[lax.rem(nloc - 1, ISN)]],
        ssem.at[lax.rem(nloc - 1, NBUF)]).wait()
    plsc.subcore_barrier()
    pltpu.sync_copy(agg_sh.at[slab], agg_hbm.at[cid, slab])

  return k(es, ed, h_scaled, zeros2)


def _tc_matmul_scale(x, w, degp4, n, d):
  grid = n // BLK

  def body(x_ref, w_ref, deg_ref, o_ref):
    deg = deg_ref[0, 0] + deg_ref[1, 0]
    norm = lax.rsqrt(jnp.maximum(deg, 1.0))
    h = jnp.dot(x_ref[...], w_ref[...], preferred_element_type=jnp.float32)
    o_ref[...] = h * norm

  return pl.pallas_call(
      body,
      grid=(grid,),
      in_specs=[
          pl.BlockSpec((BLK, d), lambda i: (i, 0)),
          pl.BlockSpec((d, d), lambda i: (0, 0)),
          pl.BlockSpec((NC, 2, BLK, 1), lambda i: (0, 0, i, 0)),
      ],
      out_specs=pl.BlockSpec((BLK, d), lambda i: (i, 0)),
      out_shape=jax.ShapeDtypeStruct((n, d), jnp.float32),
  )(x, w, degp4)


def _tc_finalize(aggp, degp4, b2, n, d):
  grid = n // BLK

  def body(agg_ref, deg_ref, b_ref, act_ref, clone_ref):
    agg = agg_ref[0] + agg_ref[1]
    deg = deg_ref[0, 1] + deg_ref[1, 1]
    norm = lax.rsqrt(jnp.maximum(deg, 1.0))
    out = agg * norm + b_ref[...]
    act = jnp.maximum(out, 0.0)
    act_ref[...] = act
    clone_ref[...] = jnp.where(act >= 0.5, 1.0, 0.0).astype(jnp.float32)

  return pl.pallas_call(
      body,
      grid=(grid,),
      in_specs=[
          pl.BlockSpec((NC, BLK, d), lambda i: (0, i, 0)),
          pl.BlockSpec((NC, 2, BLK, 1), lambda i: (0, 0, i, 0)),
          pl.BlockSpec((1, d), lambda i: (0, 0)),
      ],
      out_specs=[
          pl.BlockSpec((BLK, d), lambda i: (i, 0)),
          pl.BlockSpec((BLK, d), lambda i: (i, 0)),
      ],
      out_shape=[
          jax.ShapeDtypeStruct((n, d), jnp.float32),
          jax.ShapeDtypeStruct((n, d), jnp.float32),
      ],
  )(aggp, degp4, b2)


def kernel(in_feat, edge_index, W, b):
  n, d = in_feat.shape
  e = edge_index.shape[1]
  assert e % CHUNK == 0 and n % BLK == 0
  ncht = e // CHUNK
  assert ncht // NW >= max(NBUF, DDEPTH) + 2
  n_acc = (NS * 128) * (-(-n // (NS * 128)))

  es = edge_index[0].astype(jnp.int32)
  ed = edge_index[1].astype(jnp.int32)
  zeros1 = jnp.zeros((n_acc,), jnp.float32)
  zeros2 = jnp.zeros((n_acc, d), jnp.float32)

  degp = _sc_degrees(es, ed, zeros1, n_acc, ncht)
  degp4 = degp.reshape(NC, 2, n_acc, 1)[:, :, :n]
  h_scaled = _tc_matmul_scale(in_feat, W, degp4, n, d)
  aggp = _sc_aggregate(es, ed, h_scaled, zeros2, n_acc, ncht, d)
  h_act, h_clone = _tc_finalize(aggp, degp4, b.reshape(1, d), n, d)
  return (h_act, h_clone)


# speedup vs baseline: 1.1121x; 1.0515x over previous
import functools

import jax
import jax.numpy as jnp
from jax import lax
from jax.experimental import pallas as pl
from jax.experimental.pallas import tpu as pltpu
from jax.experimental.pallas import tpu_sc as plsc

NC = 2
NS = 16
NW = NC * NS
CHUNK = 128
NBUF = 2
ISN = 6
DDEPTH = 3
BLK = 2000

_mesh = functools.partial(
    plsc.VectorSubcoreMesh, core_axis_name="c", subcore_axis_name="s",
    num_cores=NC, num_subcores=NS)


def _chunk_range(w, ncht):
  c0 = (w * ncht) // NW
  c1 = ((w + 1) * ncht) // NW
  return c0, c1


def _sc_degrees(ef, zeros1, n_acc, ncht):
  slab_n = n_acc // NS

  @functools.partial(
      pl.kernel,
      out_type=jax.ShapeDtypeStruct((NC, 2, n_acc), jnp.float32),
      mesh=_mesh(),
      scratch_types=[
          pltpu.VMEM((ISN, CHUNK), jnp.int32),
          pltpu.VMEM((ISN, CHUNK), jnp.int32),
          pltpu.VMEM((CHUNK,), jnp.float32),
          pltpu.VMEM_SHARED((n_acc,), jnp.float32),
          pltpu.VMEM_SHARED((n_acc,), jnp.float32),
          pltpu.SemaphoreType.DMA((ISN,)),
          pltpu.SemaphoreType.DMA((ISN,)),
          pltpu.SemaphoreType.DMA((DDEPTH + 1,)),
          pltpu.SemaphoreType.DMA((DDEPTH + 1,)),
      ],
  )
  def k(ef_hbm, z_hbm, deg_hbm, sidx_v, didx_v, ones_v,
        dego_sh, degi_sh, xsem, ysem, osem, isem):
    cid = lax.axis_index("c")
    sid = lax.axis_index("s")
    w = cid * NS + sid
    c0, c1 = _chunk_range(w, ncht)
    nloc = c1 - c0
    for t in range(CHUNK // 16):
      ones_v[pl.ds(t * 16, 16)] = jnp.ones((16,), jnp.float32)
    slab = pl.ds(sid * slab_n, slab_n)
    pltpu.sync_copy(z_hbm.at[slab], dego_sh.at[slab])
    pltpu.sync_copy(z_hbm.at[slab], degi_sh.at[slab])
    plsc.subcore_barrier()

    def fire_idx(i, ib):
      off = pl.multiple_of((c0 + i) * CHUNK, CHUNK)
      pltpu.async_copy(ef_hbm.at[pl.ds(off, CHUNK)], sidx_v.at[ib],
                       xsem.at[ib])
      pltpu.async_copy(ef_hbm.at[pl.ds(ncht * CHUNK + off, CHUNK)],
                       didx_v.at[ib], ysem.at[ib])

    def wait_idx(i, ib):
      off = pl.multiple_of((c0 + i) * CHUNK, CHUNK)
      pltpu.make_async_copy(
          ef_hbm.at[pl.ds(off, CHUNK)], sidx_v.at[ib], xsem.at[ib]).wait()
      pltpu.make_async_copy(
          ef_hbm.at[pl.ds(ncht * CHUNK + off, CHUNK)], didx_v.at[ib],
          ysem.at[ib]).wait()

    def fire_sc(i):
      cb = lax.rem(i, DDEPTH + 1)
      ib = lax.rem(i, ISN)
      pltpu.async_copy(ones_v, dego_sh.at[sidx_v.at[ib]], osem.at[cb],
                       add=True)
      pltpu.async_copy(ones_v, degi_sh.at[didx_v.at[ib]], isem.at[cb],
                       add=True)

    def drain_sc(i):
      cb = lax.rem(i, DDEPTH + 1)
      ib = lax.rem(i, ISN)
      pltpu.make_async_copy(
          ones_v, dego_sh.at[sidx_v.at[ib]], osem.at[cb]).wait()
      pltpu.make_async_copy(
          ones_v, degi_sh.at[didx_v.at[ib]], isem.at[cb]).wait()

    for r in range(DDEPTH):
      fire_idx(r, r)

    def body(i, carry):
      @pl.when(i + DDEPTH < nloc)
      def _():
        fire_idx(i + DDEPTH, lax.rem(i + DDEPTH, ISN))

      wait_idx(i, lax.rem(i, ISN))
      fire_sc(i)

      @pl.when(i >= 2)
      def _():
        drain_sc(i - 2)

      return carry

    lax.fori_loop(0, nloc, body, 0)
    drain_sc(nloc - 2)
    drain_sc(nloc - 1)
    plsc.subcore_barrier()
    pltpu.sync_copy(dego_sh.at[slab], deg_hbm.at[cid, 0, slab])
    pltpu.sync_copy(degi_sh.at[slab], deg_hbm.at[cid, 1, slab])

  return k(ef, zeros1)


def _sc_aggregate(ef, h_scaled, zeros2, n_acc, ncht, d):
  slab_n = n_acc // NS

  @functools.partial(
      pl.kernel,
      out_type=jax.ShapeDtypeStruct((NC, n_acc, d), jnp.float32),
      mesh=_mesh(),
      scratch_types=[
          pltpu.VMEM((ISN, CHUNK), jnp.int32),
          pltpu.VMEM((ISN, CHUNK), jnp.int32),
          pltpu.VMEM((NBUF, CHUNK, d), jnp.float32),
          pltpu.VMEM_SHARED((n_acc, d), jnp.float32),
          pltpu.SemaphoreType.DMA((ISN,)),
          pltpu.SemaphoreType.DMA((ISN,)),
          pltpu.SemaphoreType.DMA((NBUF,)),
          pltpu.SemaphoreType.DMA((NBUF,)),
      ],
  )
  def k(ef_hbm, h_hbm, z_hbm, agg_hbm,
        sidx_v, didx_v, rows_v, agg_sh, xsem, ysem, gsem, ssem):
    cid = lax.axis_index("c")
    sid = lax.axis_index("s")
    w = cid * NS + sid
    c0, c1 = _chunk_range(w, ncht)
    nloc = c1 - c0
    slab = pl.ds(sid * slab_n, slab_n)
    pltpu.sync_copy(z_hbm.at[slab], agg_sh.at[slab])
    plsc.subcore_barrier()

    def fire_idx(i, ib):
      off = pl.multiple_of((c0 + i) * CHUNK, CHUNK)
      pltpu.async_copy(ef_hbm.at[pl.ds(off, CHUNK)], sidx_v.at[ib],
                       xsem.at[ib])
      pltpu.async_copy(ef_hbm.at[pl.ds(ncht * CHUNK + off, CHUNK)],
                       didx_v.at[ib], ysem.at[ib])

    def wait_idx(i, ib):
      off = pl.multiple_of((c0 + i) * CHUNK, CHUNK)
      pltpu.make_async_copy(
          ef_hbm.at[pl.ds(off, CHUNK)], sidx_v.at[ib], xsem.at[ib]).wait()
      pltpu.make_async_copy(
          ef_hbm.at[pl.ds(ncht * CHUNK + off, CHUNK)], didx_v.at[ib],
          ysem.at[ib]).wait()

    def fire_gather(i, bb):
      ib = lax.rem(i, ISN)
      wait_idx(i, ib)
      pltpu.async_copy(h_hbm.at[sidx_v.at[ib]], rows_v.at[bb], gsem.at[bb])

    for r in range(DDEPTH):
      fire_idx(r, r)
    for bi in range(NBUF):
      fire_gather(bi, bi)

    def body(i, carry):
      ib = lax.rem(i, ISN)
      jb = lax.rem(i, NBUF)

      @pl.when(i + DDEPTH < nloc)
      def _():
        fire_idx(i + DDEPTH, lax.rem(i + DDEPTH, ISN))

      pltpu.make_async_copy(
          h_hbm.at[sidx_v.at[ib]], rows_v.at[jb], gsem.at[jb]).wait()
      pltpu.async_copy(
          rows_v.at[jb], agg_sh.at[didx_v.at[ib]], ssem.at[jb], add=True)

      @pl.when(i >= 1)
      def _():
        ip = i - 1
        pb = lax.rem(ip, NBUF)
        pltpu.make_async_copy(
            rows_v.at[pb], agg_sh.at[didx_v.at[lax.rem(ip, ISN)]],
            ssem.at[pb]).wait()

        @pl.when(ip + NBUF < nloc)
        def _():
          fire_gather(ip + NBUF, pb)

      return carry

    lax.fori_loop(0, nloc, body, 0)
    pltpu.make_async_copy(
        rows_v.at[lax.rem(nloc - 1, NBUF)],
        agg_sh.at[didx_v.at[lax.rem(nloc - 1, ISN)]],
        ssem.at[lax.rem(nloc - 1, NBUF)]).wait()
    plsc.subcore_barrier()
    pltpu.sync_copy(agg_sh.at[slab], agg_hbm.at[cid, slab])

  return k(ef, h_scaled, zeros2)


def _tc_matmul_scale(x, w, degp4, n, d):
  grid = n // BLK

  def body(x_ref, w_ref, deg_ref, o_ref):
    deg = deg_ref[0, 0] + deg_ref[1, 0]
    norm = lax.rsqrt(jnp.maximum(deg, 1.0))
    h = jnp.dot(x_ref[...], w_ref[...], preferred_element_type=jnp.float32)
    o_ref[...] = h * norm

  return pl.pallas_call(
      body,
      grid=(grid,),
      in_specs=[
          pl.BlockSpec((BLK, d), lambda i: (i, 0)),
          pl.BlockSpec((d, d), lambda i: (0, 0)),
          pl.BlockSpec((NC, 2, BLK, 1), lambda i: (0, 0, i, 0)),
      ],
      out_specs=pl.BlockSpec((BLK, d), lambda i: (i, 0)),
      out_shape=jax.ShapeDtypeStruct((n, d), jnp.float32),
  )(x, w, degp4)


def _tc_finalize(aggp, degp4, b2, n, d):
  grid = n // BLK

  def body(agg_ref, deg_ref, b_ref, act_ref, clone_ref):
    agg = agg_ref[0] + agg_ref[1]
    deg = deg_ref[0, 1] + deg_ref[1, 1]
    norm = lax.rsqrt(jnp.maximum(deg, 1.0))
    out = agg * norm + b_ref[...]
    act = jnp.maximum(out, 0.0)
    act_ref[...] = act
    clone_ref[...] = jnp.where(act >= 0.5, 1.0, 0.0).astype(jnp.float32)

  return pl.pallas_call(
      body,
      grid=(grid,),
      in_specs=[
          pl.BlockSpec((NC, BLK, d), lambda i: (0, i, 0)),
          pl.BlockSpec((NC, 2, BLK, 1), lambda i: (0, 0, i, 0)),
          pl.BlockSpec((1, d), lambda i: (0, 0)),
      ],
      out_specs=[
          pl.BlockSpec((BLK, d), lambda i: (i, 0)),
          pl.BlockSpec((BLK, d), lambda i: (i, 0)),
      ],
      out_shape=[
          jax.ShapeDtypeStruct((n, d), jnp.float32),
          jax.ShapeDtypeStruct((n, d), jnp.float32),
      ],
  )(aggp, degp4, b2)


def kernel(in_feat, edge_index, W, b):
  n, d = in_feat.shape
  e = edge_index.shape[1]
  assert e % CHUNK == 0 and n % BLK == 0
  ncht = e // CHUNK
  assert ncht // NW >= max(NBUF, DDEPTH) + 2
  n_acc = (NS * 128) * (-(-n // (NS * 128)))

  ef = edge_index.astype(jnp.int32).reshape(2 * e)
  zeros1 = jnp.zeros((n_acc,), jnp.float32)
  zeros2 = jnp.zeros((n_acc, d), jnp.float32)

  degp = _sc_degrees(ef, zeros1, n_acc, ncht)
  degp4 = degp.reshape(NC, 2, n_acc, 1)[:, :, :n]
  h_scaled = _tc_matmul_scale(in_feat, W, degp4, n, d)
  aggp = _sc_aggregate(ef, h_scaled, zeros2, n_acc, ncht, d)
  h_act, h_clone = _tc_finalize(aggp, degp4, b.reshape(1, d), n, d)
  return (h_act, h_clone)


# speedup vs baseline: 1.1385x; 1.0237x over previous
import functools

import jax
import jax.numpy as jnp
from jax import lax
from jax.experimental import pallas as pl
from jax.experimental.pallas import tpu as pltpu
from jax.experimental.pallas import tpu_sc as plsc

NC = 2
NS = 16
NW = NC * NS
CHUNK = 128
NBUF = 2
ISN = 6
DDEPTH = 3
BLK = 2000

_mesh = functools.partial(
    plsc.VectorSubcoreMesh, core_axis_name="c", subcore_axis_name="s",
    num_cores=NC, num_subcores=NS)


def _chunk_range(w, ncht):
  c0 = (w * ncht) // NW
  c1 = ((w + 1) * ncht) // NW
  return c0, c1


def _sc_degree1(ef, zeros1, n_acc, ncht, which):
  slab_n = n_acc // NS
  base = which * ncht * CHUNK

  @functools.partial(
      pl.kernel,
      out_type=jax.ShapeDtypeStruct((NC, 1, n_acc), jnp.float32),
      mesh=_mesh(),
      scratch_types=[
          pltpu.VMEM((ISN, CHUNK), jnp.int32),
          pltpu.VMEM((CHUNK,), jnp.float32),
          pltpu.VMEM_SHARED((n_acc,), jnp.float32),
          pltpu.SemaphoreType.DMA((ISN,)),
          pltpu.SemaphoreType.DMA((DDEPTH + 1,)),
      ],
      name=f"degrees{which}",
  )
  def k(ef_hbm, z_hbm, deg_hbm, idx_v, ones_v, deg_sh, xsem, osem):
    cid = lax.axis_index("c")
    sid = lax.axis_index("s")
    w = cid * NS + sid
    c0, c1 = _chunk_range(w, ncht)
    nloc = c1 - c0
    for t in range(CHUNK // 16):
      ones_v[pl.ds(t * 16, 16)] = jnp.ones((16,), jnp.float32)
    slab = pl.ds(sid * slab_n, slab_n)
    pltpu.sync_copy(z_hbm.at[slab], deg_sh.at[slab])
    plsc.subcore_barrier()

    def fire_idx(i, ib):
      off = pl.multiple_of(base + (c0 + i) * CHUNK, CHUNK)
      pltpu.async_copy(ef_hbm.at[pl.ds(off, CHUNK)], idx_v.at[ib],
                       xsem.at[ib])

    def wait_idx(i, ib):
      off = pl.multiple_of(base + (c0 + i) * CHUNK, CHUNK)
      pltpu.make_async_copy(
          ef_hbm.at[pl.ds(off, CHUNK)], idx_v.at[ib], xsem.at[ib]).wait()

    def fire_sc(i):
      pltpu.async_copy(ones_v, deg_sh.at[idx_v.at[lax.rem(i, ISN)]],
                       osem.at[lax.rem(i, DDEPTH + 1)], add=True)

    def drain_sc(i):
      pltpu.make_async_copy(
          ones_v, deg_sh.at[idx_v.at[lax.rem(i, ISN)]],
          osem.at[lax.rem(i, DDEPTH + 1)]).wait()

    for r in range(DDEPTH):
      fire_idx(r, r)

    def body(i, carry):
      @pl.when(i + DDEPTH < nloc)
      def _():
        fire_idx(i + DDEPTH, lax.rem(i + DDEPTH, ISN))

      wait_idx(i, lax.rem(i, ISN))
      fire_sc(i)

      @pl.when(i >= 2)
      def _():
        drain_sc(i - 2)

      return carry

    lax.fori_loop(0, nloc, body, 0)
    drain_sc(nloc - 2)
    drain_sc(nloc - 1)
    plsc.subcore_barrier()
    pltpu.sync_copy(deg_sh.at[slab], deg_hbm.at[cid, 0, slab])

  return k(ef, zeros1)


def _sc_aggregate(ef, h_scaled, zeros2, n_acc, ncht, d):
  slab_n = n_acc // NS

  @functools.partial(
      pl.kernel,
      out_type=jax.ShapeDtypeStruct((NC, n_acc, d), jnp.float32),
      mesh=_mesh(),
      scratch_types=[
          pltpu.VMEM((ISN, CHUNK), jnp.int32),
          pltpu.VMEM((ISN, CHUNK), jnp.int32),
          pltpu.VMEM((NBUF, CHUNK, d), jnp.float32),
          pltpu.VMEM_SHARED((n_acc, d), jnp.float32),
          pltpu.SemaphoreType.DMA((ISN,)),
          pltpu.SemaphoreType.DMA((ISN,)),
          pltpu.SemaphoreType.DMA((NBUF,)),
          pltpu.SemaphoreType.DMA((NBUF,)),
      ],
  )
  def k(ef_hbm, h_hbm, z_hbm, agg_hbm,
        sidx_v, didx_v, rows_v, agg_sh, xsem, ysem, gsem, ssem):
    cid = lax.axis_index("c")
    sid = lax.axis_index("s")
    w = cid * NS + sid
    c0, c1 = _chunk_range(w, ncht)
    nloc = c1 - c0
    slab = pl.ds(sid * slab_n, slab_n)
    pltpu.sync_copy(z_hbm.at[slab], agg_sh.at[slab])
    plsc.subcore_barrier()

    def fire_idx(i, ib):
      off = pl.multiple_of((c0 + i) * CHUNK, CHUNK)
      pltpu.async_copy(ef_hbm.at[pl.ds(off, CHUNK)], sidx_v.at[ib],
                       xsem.at[ib])
      pltpu.async_copy(ef_hbm.at[pl.ds(ncht * CHUNK + off, CHUNK)],
                       didx_v.at[ib], ysem.at[ib])

    def wait_idx(i, ib):
      off = pl.multiple_of((c0 + i) * CHUNK, CHUNK)
      pltpu.make_async_copy(
          ef_hbm.at[pl.ds(off, CHUNK)], sidx_v.at[ib], xsem.at[ib]).wait()
      pltpu.make_async_copy(
          ef_hbm.at[pl.ds(ncht * CHUNK + off, CHUNK)], didx_v.at[ib],
          ysem.at[ib]).wait()

    def fire_gather(i, bb):
      ib = lax.rem(i, ISN)
      wait_idx(i, ib)
      pltpu.async_copy(h_hbm.at[sidx_v.at[ib]], rows_v.at[bb], gsem.at[bb])

    for r in range(DDEPTH):
      fire_idx(r, r)
    for bi in range(NBUF):
      fire_gather(bi, bi)

    def body(i, carry):
      ib = lax.rem(i, ISN)
      jb = lax.rem(i, NBUF)

      @pl.when(i + DDEPTH < nloc)
      def _():
        fire_idx(i + DDEPTH, lax.rem(i + DDEPTH, ISN))

      pltpu.make_async_copy(
          h_hbm.at[sidx_v.at[ib]], rows_v.at[jb], gsem.at[jb]).wait()
      pltpu.async_copy(
          rows_v.at[jb], agg_sh.at[didx_v.at[ib]], ssem.at[jb], add=True)

      @pl.when(i >= 1)
      def _():
        ip = i - 1
        pb = lax.rem(ip, NBUF)
        pltpu.make_async_copy(
            rows_v.at[pb], agg_sh.at[didx_v.at[lax.rem(ip, ISN)]],
            ssem.at[pb]).wait()

        @pl.when(ip + NBUF < nloc)
        def _():
          fire_gather(ip + NBUF, pb)

      return carry

    lax.fori_loop(0, nloc, body, 0)
    pltpu.make_async_copy(
        rows_v.at[lax.rem(nloc - 1, NBUF)],
        agg_sh.at[didx_v.at[lax.rem(nloc - 1, ISN)]],
        ssem.at[lax.rem(nloc - 1, NBUF)]).wait()
    plsc.subcore_barrier()
    pltpu.sync_copy(agg_sh.at[slab], agg_hbm.at[cid, slab])

  return k(ef, h_scaled, zeros2)


def _tc_matmul_scale(x, w, degp4, n, d):
  grid = n // BLK

  def body(x_ref, w_ref, deg_ref, o_ref):
    deg = deg_ref[0] + deg_ref[1]
    norm = lax.rsqrt(jnp.maximum(deg, 1.0))
    h = jnp.dot(x_ref[...], w_ref[...], preferred_element_type=jnp.float32)
    o_ref[...] = h * norm

  return pl.pallas_call(
      body,
      grid=(grid,),
      in_specs=[
          pl.BlockSpec((BLK, d), lambda i: (i, 0)),
          pl.BlockSpec((d, d), lambda i: (0, 0)),
          pl.BlockSpec((NC, BLK, 1), lambda i: (0, i, 0)),
      ],
      out_specs=pl.BlockSpec((BLK, d), lambda i: (i, 0)),
      out_shape=jax.ShapeDtypeStruct((n, d), jnp.float32),
  )(x, w, degp4)


def _tc_finalize(aggp, degp4, b2, n, d):
  grid = n // BLK

  def body(agg_ref, deg_ref, b_ref, act_ref, clone_ref):
    agg = agg_ref[0] + agg_ref[1]
    deg = deg_ref[0] + deg_ref[1]
    norm = lax.rsqrt(jnp.maximum(deg, 1.0))
    out = agg * norm + b_ref[...]
    act = jnp.maximum(out, 0.0)
    act_ref[...] = act
    clone_ref[...] = jnp.where(act >= 0.5, 1.0, 0.0).astype(jnp.float32)

  return pl.pallas_call(
      body,
      grid=(grid,),
      in_specs=[
          pl.BlockSpec((NC, BLK, d), lambda i: (0, i, 0)),
          pl.BlockSpec((NC, BLK, 1), lambda i: (0, i, 0)),
          pl.BlockSpec((1, d), lambda i: (0, 0)),
      ],
      out_specs=[
          pl.BlockSpec((BLK, d), lambda i: (i, 0)),
          pl.BlockSpec((BLK, d), lambda i: (i, 0)),
      ],
      out_shape=[
          jax.ShapeDtypeStruct((n, d), jnp.float32),
          jax.ShapeDtypeStruct((n, d), jnp.float32),
      ],
  )(aggp, degp4, b2)


def kernel(in_feat, edge_index, W, b):
  n, d = in_feat.shape
  e = edge_index.shape[1]
  assert e % CHUNK == 0 and n % BLK == 0
  ncht = e // CHUNK
  assert ncht // NW >= max(NBUF, DDEPTH) + 2
  n_acc = (NS * 128) * (-(-n // (NS * 128)))

  ef = edge_index.astype(jnp.int32).reshape(2 * e)
  zeros1 = jnp.zeros((n_acc,), jnp.float32)
  zeros2 = jnp.zeros((n_acc, d), jnp.float32)

  dego = _sc_degree1(ef, zeros1, n_acc, ncht, 0)
  degi = _sc_degree1(ef, zeros1, n_acc, ncht, 1)
  dego4 = dego.reshape(NC, n_acc, 1)[:, :n]
  degi4 = degi.reshape(NC, n_acc, 1)[:, :n]
  h_scaled = _tc_matmul_scale(in_feat, W, dego4, n, d)
  aggp = _sc_aggregate(ef, h_scaled, zeros2, n_acc, ncht, d)
  h_act, h_clone = _tc_finalize(aggp, degi4, b.reshape(1, d), n, d)
  return (h_act, h_clone)
